# baseline (speedup 1.0000x reference)
import functools

import jax
import jax.numpy as jnp
from jax import lax
from jax.experimental import pallas as pl
from jax.experimental.pallas import tpu as pltpu
from jax.experimental.pallas import tpu_sc as plsc

B = 16384
D = 64
NC = 2
NS = 16
NW = NC * NS
B_PER_W = B // NW
L = 16


def _body(u_hbm, i_hbm, ue_hbm, ie_hbm, out_hbm,
          u_idx, i_idx, ue_rows, ie_rows, out_v, sem_u, sem_i):
    wid = lax.axis_index("s") * NC + lax.axis_index("c")
    base = wid * B_PER_W

    pltpu.sync_copy(u_hbm.at[pl.ds(base, B_PER_W)], u_idx)
    pltpu.sync_copy(i_hbm.at[pl.ds(base, B_PER_W)], i_idx)

    cp_u = pltpu.make_async_copy(ue_hbm.at[u_idx], ue_rows, sem_u)
    cp_i = pltpu.make_async_copy(ie_hbm.at[i_idx], ie_rows, sem_i)
    cp_u.start()
    cp_i.start()
    cp_u.wait()
    cp_i.wait()

    lane = lax.iota(jnp.int32, L)

    def group(g, carry):
        acc = jnp.zeros((L,), jnp.float32)
        for bb in range(L):
            b = g * L + bb
            prod = ue_rows[b, pl.ds(0, L)] * ie_rows[b, pl.ds(0, L)]
            for c in range(1, D // L):
                prod = prod + (ue_rows[b, pl.ds(c * L, L)]
                               * ie_rows[b, pl.ds(c * L, L)])
            acc = jnp.where(lane == bb, jnp.sum(prod), acc)
        out_v[pl.ds(g * L, L)] = acc
        return carry

    lax.fori_loop(0, B_PER_W // L, group, 0)

    pltpu.sync_copy(out_v, out_hbm.at[pl.ds(base, B_PER_W)])


@jax.jit
def _score(u, i, user_emb, item_emb):
    mesh = plsc.VectorSubcoreMesh(core_axis_name="c", subcore_axis_name="s")
    f = functools.partial(
        pl.kernel,
        out_type=jax.ShapeDtypeStruct((B,), jnp.float32),
        mesh=mesh,
        compiler_params=pltpu.CompilerParams(
            needs_layout_passes=False, use_tc_tiling_on_sc=False),
        scratch_types=[
            pltpu.VMEM((B_PER_W,), jnp.int32),
            pltpu.VMEM((B_PER_W,), jnp.int32),
            pltpu.VMEM((B_PER_W, D), jnp.float32),
            pltpu.VMEM((B_PER_W, D), jnp.float32),
            pltpu.VMEM((B_PER_W,), jnp.float32),
            pltpu.SemaphoreType.DMA,
            pltpu.SemaphoreType.DMA,
        ],
    )(_body)
    return f(u, i, user_emb, item_emb)


def kernel(u, i, user_emb, item_emb):
    return _score(u, i, user_emb, item_emb)
